# baseline (speedup 1.0000x reference)
import functools
import jax
import jax.numpy as jnp
from jax import lax
from jax.experimental import pallas as pl
from jax.experimental.pallas import tpu as pltpu
from jax.experimental.pallas import tpu_sc as plsc

_CODEBOOK = 1024
_D = 64
_COMMIT = 0.25


def _argmin_body(z_ref, emb_ref, idx_ref, loss_ref):
    b = pl.program_id(0)
    zb = z_ref[0]
    emb = emb_ref[...]
    zt = zb.T
    zsq = jnp.sum(zt * zt, axis=1)
    esq = jnp.sum(emb * emb, axis=1)
    scores = jax.lax.dot_general(
        zt, emb, (((1,), (1,)), ((), ())),
        preferred_element_type=jnp.float32)
    dist = (zsq[:, None] - 2.0 * scores) + esq[None, :]
    m = jnp.min(dist, axis=1, keepdims=True)
    c_iota = jax.lax.broadcasted_iota(jnp.int32, dist.shape, 1)
    idx = jnp.min(jnp.where(dist == m, c_iota, _CODEBOOK), axis=1)
    idx_ref[0, 0, :] = idx
    part = jnp.sum(m)

    @pl.when(b == 0)
    def _init():
        loss_ref[0, 0] = jnp.float32(0.0)

    loss_ref[0, 0] += part


_NC = 2
_NS = 16
_NW = _NC * _NS
_DSPLIT = 2
_DSUB = _D // _DSPLIT
_L = 16


def _make_gather(B, P):
    assert B * _DSPLIT == _NW
    mesh = plsc.VectorSubcoreMesh(core_axis_name="c", subcore_axis_name="s")

    @functools.partial(
        pl.kernel, mesh=mesh,
        compiler_params=pltpu.CompilerParams(
            use_tc_tiling_on_sc=False, needs_layout_passes=False),
        out_type=jax.ShapeDtypeStruct((B, _D, P), jnp.float32),
        scratch_types=[
            pltpu.VMEM((P,), jnp.int32),
            pltpu.VMEM((_DSUB, _CODEBOOK), jnp.float32),
            pltpu.VMEM((_DSUB, P), jnp.float32),
        ],
    )
    def gather_k(embt_hbm, idx_hbm, out_hbm, idx_v, tab_v, out_v):
        wid = lax.axis_index("s") * _NC + lax.axis_index("c")
        b = wid % B
        dlo = (wid // B) * _DSUB
        pltpu.sync_copy(idx_hbm.at[b], idx_v)
        pltpu.sync_copy(embt_hbm.at[pl.ds(dlo, _DSUB)], tab_v)

        def step(j, carry):
            col = idx_v[pl.ds(j * _L, _L)]
            for d in range(_DSUB):
                row = jnp.full((_L,), d, jnp.int32)
                out_v[d, pl.ds(j * _L, _L)] = plsc.load_gather(
                    tab_v, [row, col])
            return carry

        lax.fori_loop(0, P // _L, step, 0)
        pltpu.sync_copy(out_v, out_hbm.at[b, pl.ds(dlo, _DSUB)])

    return gather_k


def kernel(z, embedding):
    B, D, H, W = z.shape
    P = H * W
    N = B * P
    z3 = z.reshape(B, D, P)

    idx3, loss_raw = pl.pallas_call(
        _argmin_body,
        grid=(B,),
        in_specs=[
            pl.BlockSpec((1, D, P), lambda b: (b, 0, 0)),
            pl.BlockSpec((_CODEBOOK, D), lambda b: (0, 0)),
        ],
        out_specs=[
            pl.BlockSpec((1, 1, P), lambda b: (b, 0, 0)),
            pl.BlockSpec((1, 1), lambda b: (0, 0),
                         memory_space=pltpu.SMEM),
        ],
        out_shape=[
            jax.ShapeDtypeStruct((B, 1, P), jnp.int32),
            jax.ShapeDtypeStruct((1, 1), jnp.float32),
        ],
    )(z3, embedding)

    emb_t = embedding.T
    zq3 = _make_gather(B, P)(emb_t, idx3.reshape(B, P))

    z_q = zq3.reshape(B, D, H, W)
    indices = idx3.reshape(B, H, W)
    loss = loss_raw[0, 0] * (_COMMIT / (N * D))
    return (z_q, loss, indices)


# speedup vs baseline: 1.1900x; 1.1900x over previous
import jax
import jax.numpy as jnp
from jax.experimental import pallas as pl
from jax.experimental.pallas import tpu as pltpu

_CODEBOOK = 1024
_D = 64
_COMMIT = 0.25


def _vq_body(z_ref, emb_ref, zq_ref, idx_ref, loss_ref):
    b = pl.program_id(0)
    zb = z_ref[0]
    emb = emb_ref[...]
    zt = zb.T
    zsq = jnp.sum(zt * zt, axis=1)
    esq = jnp.sum(emb * emb, axis=1)
    scores = jax.lax.dot_general(
        zt, emb, (((1,), (1,)), ((), ())),
        preferred_element_type=jnp.float32)
    dist = (zsq[:, None] - 2.0 * scores) + esq[None, :]
    m = jnp.min(dist, axis=1, keepdims=True)
    c_iota = jax.lax.broadcasted_iota(jnp.int32, dist.shape, 1)
    idx = jnp.min(jnp.where(dist == m, c_iota, _CODEBOOK), axis=1)
    idx_ref[0, 0, :] = idx

    onehot = (c_iota == idx[:, None]).astype(jnp.bfloat16)
    hi = emb.astype(jnp.bfloat16)
    lo = (emb - hi.astype(jnp.float32)).astype(jnp.bfloat16)
    zq = jax.lax.dot_general(
        hi, onehot, (((0,), (1,)), ((), ())),
        preferred_element_type=jnp.float32)
    zq += jax.lax.dot_general(
        lo, onehot, (((0,), (1,)), ((), ())),
        preferred_element_type=jnp.float32)
    zq_ref[0] = zq

    part = jnp.sum(m)

    @pl.when(b == 0)
    def _init():
        loss_ref[0, 0] = jnp.float32(0.0)

    loss_ref[0, 0] += part


def kernel(z, embedding):
    B, D, H, W = z.shape
    P = H * W
    z3 = z.reshape(B, D, P)

    zq3, idx3, loss_raw = pl.pallas_call(
        _vq_body,
        grid=(B,),
        in_specs=[
            pl.BlockSpec((1, D, P), lambda b: (b, 0, 0)),
            pl.BlockSpec((_CODEBOOK, D), lambda b: (0, 0)),
        ],
        out_specs=[
            pl.BlockSpec((1, D, P), lambda b: (b, 0, 0)),
            pl.BlockSpec((1, 1, P), lambda b: (b, 0, 0)),
            pl.BlockSpec((1, 1), lambda b: (0, 0),
                         memory_space=pltpu.SMEM),
        ],
        out_shape=[
            jax.ShapeDtypeStruct((B, D, P), jnp.float32),
            jax.ShapeDtypeStruct((B, 1, P), jnp.int32),
            jax.ShapeDtypeStruct((1, 1), jnp.float32),
        ],
    )(z3, embedding)

    z_q = zq3.reshape(B, D, H, W)
    indices = idx3.reshape(B, H, W)
    loss = loss_raw[0, 0] * (_COMMIT / (B * P * D))
    return (z_q, loss, indices)


# speedup vs baseline: 1.3006x; 1.0930x over previous
import jax
import jax.numpy as jnp
from jax.experimental import pallas as pl
from jax.experimental.pallas import tpu as pltpu

_CODEBOOK = 1024
_D = 64
_COMMIT = 0.25


def _vq_body(z_ref, emb_ref, zq_ref, idx_ref, loss_ref):
    b = pl.program_id(0)
    zb = z_ref[0]
    emb = emb_ref[...]
    zt = zb.T
    zsq = jnp.sum(zt * zt, axis=1)
    esq = jnp.sum(emb * emb, axis=1)
    neg2s = jax.lax.dot_general(
        -2.0 * zt, emb, (((1,), (1,)), ((), ())),
        preferred_element_type=jnp.float32)
    dist = (zsq[:, None] + neg2s) + esq[None, :]
    m = jnp.min(dist, axis=1, keepdims=True)
    c_iota = jax.lax.broadcasted_iota(
        jnp.int32, dist.shape, 1).astype(jnp.float32)
    idxf = jnp.min(jnp.where(dist == m, c_iota, jnp.float32(_CODEBOOK)),
                   axis=1)
    idx_ref[0, 0, :] = idxf.astype(jnp.int32)

    onehot = (c_iota == idxf[:, None]).astype(jnp.bfloat16)
    hi = emb.astype(jnp.bfloat16)
    lo = (emb - hi.astype(jnp.float32)).astype(jnp.bfloat16)
    zq = jax.lax.dot_general(
        hi, onehot, (((0,), (1,)), ((), ())),
        preferred_element_type=jnp.float32)
    zq += jax.lax.dot_general(
        lo, onehot, (((0,), (1,)), ((), ())),
        preferred_element_type=jnp.float32)
    zq_ref[0] = zq

    part = jnp.sum(m)

    @pl.when(b == 0)
    def _init():
        loss_ref[0, 0] = jnp.float32(0.0)

    loss_ref[0, 0] += part


def kernel(z, embedding):
    B, D, H, W = z.shape
    P = H * W
    z3 = z.reshape(B, D, P)

    zq3, idx3, loss_raw = pl.pallas_call(
        _vq_body,
        grid=(B,),
        in_specs=[
            pl.BlockSpec((1, D, P), lambda b: (b, 0, 0)),
            pl.BlockSpec((_CODEBOOK, D), lambda b: (0, 0)),
        ],
        out_specs=[
            pl.BlockSpec((1, D, P), lambda b: (b, 0, 0)),
            pl.BlockSpec((1, 1, P), lambda b: (b, 0, 0)),
            pl.BlockSpec((1, 1), lambda b: (0, 0),
                         memory_space=pltpu.SMEM),
        ],
        out_shape=[
            jax.ShapeDtypeStruct((B, D, P), jnp.float32),
            jax.ShapeDtypeStruct((B, 1, P), jnp.int32),
            jax.ShapeDtypeStruct((1, 1), jnp.float32),
        ],
    )(z3, embedding)

    z_q = zq3.reshape(B, D, H, W)
    indices = idx3.reshape(B, H, W)
    loss = loss_raw[0, 0] * (_COMMIT / (B * P * D))
    return (z_q, loss, indices)
